# baseline (speedup 1.0000x reference)
import dataclasses

import jax
import jax.numpy as jnp
from jax import lax
from jax.experimental import pallas as pl
from jax.experimental.pallas import tpu as pltpu
from jax.experimental.pallas import tpu_sc as plsc

N = 50000
E = 800000
NS = 16
SH = 9
TP = SH * NS

NC = 2
NSC = 16

GW = 256
G_TOT = 1638400
SW = 128
E_S = 819200
ACC8 = 6400
ACCW = ACC8 // NSC
BE = 2000
BN = 5000

_vmesh = plsc.VectorSubcoreMesh(core_axis_name="core", subcore_axis_name="subcore")


def _gather_body(x_hbm, idx_hbm, out_hbm, ob):
    def body(i_vmem, o_vmem):
        pltpu.sync_copy(x_hbm.at[i_vmem.at[0]], ob)

        @pl.loop(0, GW)
        def _(j):
            o_vmem[j, :] = ob[j, pl.ds(0, NS)]

    pltpu.emit_pipeline(
        body,
        grid=(G_TOT // GW,),
        in_specs=[pl.BlockSpec((1, GW), lambda i: (0, i))],
        out_specs=[pl.BlockSpec((GW, NS), lambda i: (i, 0))],
        core_axis_name=("core", "subcore"),
        dimension_semantics=(pltpu.PARALLEL,),
    )(idx_hbm, out_hbm)


def _sc_gather(x128, idxg):
    k = pl.kernel(
        _gather_body,
        out_type=jax.ShapeDtypeStruct((G_TOT, NS), jnp.float32),
        mesh=_vmesh,
        scratch_types=[pltpu.VMEM((GW, 128), jnp.float32)],
    )
    return k(x128, idxg)


def _dense_body(ea_ref, xs_ref, xd_ref, sh_ref, dc_ref, w1_ref, b1_ref,
                w2_ref, b2_ref, wo_ref, bo_ref, msg_ref):
    xs = xs_ref[...]
    sh = sh_ref[...]
    ef = jnp.concatenate([ea_ref[...], xs, xd_ref[...]], axis=1)
    h = jnp.dot(ef, w1_ref[...], preferred_element_type=jnp.float32)
    h = jax.nn.relu(h + b1_ref[...])
    tpw = jnp.dot(h, w2_ref[...], preferred_element_type=jnp.float32)
    tpw = tpw + b2_ref[...]
    u = jnp.concatenate(
        [xs * sh[:, s:s + 1] * tpw[:, s * NS:(s + 1) * NS] for s in range(SH)],
        axis=1)
    msg = jnp.dot(u, wo_ref[...], preferred_element_type=jnp.float32)
    deg_scale = (float(N) / float(E)) ** 0.5
    msg = (msg + bo_ref[...]) * deg_scale
    k1 = dc_ref[...][:, :1]
    li = lax.broadcasted_iota(jnp.int32, (BE, 128), 1)
    rel = li - k1
    mask = (rel >= 0) & (rel < NS)
    m8 = jnp.concatenate([msg] * 8, axis=1)
    msg_ref[...] = jnp.where(mask, m8, 0.0)


def _tc_dense(gathered, edge_attr, edge_sh, dcol, W1, b1, W2, b2, Wout, bout):
    nblk = E // BE
    full = lambda shape: pl.BlockSpec(shape, lambda i: (0, 0))
    return pl.pallas_call(
        _dense_body,
        grid=(nblk,),
        in_specs=[
            pl.BlockSpec((BE, NS), lambda i: (i, 0)),
            pl.BlockSpec((BE, NS), lambda i: (i, 0)),
            pl.BlockSpec((BE, NS), lambda i: (i + nblk, 0)),
            pl.BlockSpec((BE, SH), lambda i: (i, 0)),
            pl.BlockSpec((BE, NS), lambda i: (i, 0)),
            full((3 * NS, 3 * NS)),
            full((1, 3 * NS)),
            full((3 * NS, TP)),
            full((1, TP)),
            full((TP, NS)),
            full((1, NS)),
        ],
        out_specs=pl.BlockSpec((BE, 128), lambda i: (i, 0)),
        out_shape=jax.ShapeDtypeStruct((E_S, 128), jnp.float32),
    )(edge_attr, gathered, gathered, edge_sh, dcol, W1, b1.reshape(1, -1), W2,
      b2.reshape(1, -1), Wout, bout.reshape(1, -1))


def _scatter_body(msg_hbm, idx_hbm, out_hbm, acc, zbuf):
    sid = lax.axis_index("subcore")
    cid = lax.axis_index("core")

    @pl.loop(0, SW)
    def _(j):
        @pl.loop(0, 8)
        def _(q):
            zbuf[j, pl.ds(q * NS, NS)] = jnp.zeros((NS,), jnp.float32)

    for z in range(ACCW // 80):
        pltpu.sync_copy(zbuf.at[pl.ds(0, 80)],
                        acc.at[pl.ds(sid * ACCW + z * 80, 80)])
    plsc.subcore_barrier()

    def body(m_vmem, i_vmem):
        pltpu.sync_copy(m_vmem, acc.at[i_vmem.at[0]], add=True)

    pltpu.emit_pipeline(
        body,
        grid=(E_S // SW,),
        in_specs=[
            pl.BlockSpec((SW, 128), lambda i: (i, 0)),
            pl.BlockSpec((1, SW), lambda i: (0, i)),
        ],
        out_specs=[],
        core_axis_name=("core", "subcore"),
        dimension_semantics=(pltpu.PARALLEL,),
    )(msg_hbm, idx_hbm)

    plsc.subcore_barrier()
    pltpu.sync_copy(acc.at[pl.ds(sid * ACCW, ACCW)],
                    out_hbm.at[cid, pl.ds(sid * ACCW, ACCW)])


def _sc_scatter(msg, idxs8):
    k = pl.kernel(
        _scatter_body,
        out_type=jax.ShapeDtypeStruct((NC, ACC8, 128), jnp.float32),
        mesh=_vmesh,
        scratch_types=[
            pltpu.VMEM_SHARED((ACC8, 128), jnp.float32),
            pltpu.VMEM((SW, 128), jnp.float32),
        ],
    )
    return k(msg, idxs8)


def _combine_body(x_ref, p0_ref, p1_ref, o_ref):
    o_ref[...] = x_ref[...] + p0_ref[0] + p1_ref[0]


def _tc_combine(x, pu):
    return pl.pallas_call(
        _combine_body,
        grid=(N // BN,),
        in_specs=[
            pl.BlockSpec((BN, NS), lambda i: (i, 0)),
            pl.BlockSpec((1, BN, NS), lambda i: (0, i, 0)),
            pl.BlockSpec((1, BN, NS), lambda i: (1, i, 0)),
        ],
        out_specs=pl.BlockSpec((BN, NS), lambda i: (i, 0)),
        out_shape=jax.ShapeDtypeStruct((N, NS), jnp.float32),
    )(x, pu, pu)


@jax.jit
def kernel(x, edge_index, edge_attr, edge_sh, W1, b1, W2, b2, Wout, bout):
    src = edge_index[0]
    dst = edge_index[1]
    x128 = jnp.pad(x, ((0, 0), (0, 128 - NS)))
    idxg = jnp.concatenate(
        [src, dst, jnp.zeros((G_TOT - 2 * E,), jnp.int32)]).reshape(1, G_TOT)
    dst_pad = jnp.concatenate([dst, jnp.full((E_S - E,), 8 * 6250, jnp.int32)])
    idxs8 = (dst_pad // 8).reshape(1, E_S)
    dcol = jnp.broadcast_to(((dst_pad % 8) * NS)[:, None], (E_S, NS))
    gathered = _sc_gather(x128, idxg)
    msg = _tc_dense(gathered, edge_attr, edge_sh, dcol, W1, b1, W2, b2, Wout,
                    bout)
    partials = _sc_scatter(msg, idxs8)
    pu = partials.reshape(NC, ACC8 * 8, NS)
    return _tc_combine(x, pu)
